# baseline (speedup 1.0000x reference)
import functools

import jax
import jax.numpy as jnp
from jax import lax
from jax.experimental import pallas as pl
from jax.experimental.pallas import tpu as pltpu
from jax.experimental.pallas import tpu_sc as plsc

N = 50000
E = 800000
D = 128
H = 64
G = 128

NC = 2
NS = 16
LANES = 16

HALF = N // NC
ACC = 25088
ROWS_PER_TEC = ACC // NS
ZCH = 112
CH = 128
EPT = 50176
EP = EPT * NS
NCHUNK = EPT // CH

_sc_mesh = lambda: plsc.VectorSubcoreMesh(core_axis_name="c", subcore_axis_name="s")
_SC_PARAMS = pltpu.CompilerParams(use_tc_tiling_on_sc=False)
_SC_PARAMS_NL = pltpu.CompilerParams(use_tc_tiling_on_sc=False,
                                     needs_layout_passes=False)


BE = 512
NBLKE = EPT // BE
CPB = BE // CH
DW = 16


SL = ACC // NS


def _deg_body(dst_hbm, out_hbm, dstg0, dstg1, degv, stg, parts, isem0, isem1):
    c = lax.axis_index("c")
    s = lax.axis_index("s")
    base = c * HALF
    ebase = s * EPT
    dstg = (dstg0, dstg1)
    isem = (isem0, isem1)

    @pl.loop(0, ACC // LANES)
    def _z(i):
        degv[pl.ds(i * LANES, LANES)] = jnp.zeros((LANES,), jnp.float32)

    pltpu.sync_copy(dst_hbm.at[pl.ds(ebase, BE)], dstg0)
    pltpu.async_copy(dst_hbm.at[pl.ds(ebase + BE, BE)], dstg1, isem1)

    ones_v = jnp.ones((LANES,), jnp.float32)

    @pl.loop(0, NBLKE // 2)
    def _blocks(m):
        for b in range(2):
            k = 2 * m + b

            @pl.when(k > 0)
            def _():
                pltpu.make_async_copy(dst_hbm.at[pl.ds(ebase + k * BE, BE)],
                                      dstg[b], isem[b]).wait()

            for g in range(BE // LANES):
                v = dstg[b][pl.ds(g * LANES, LANES)]
                loc = v - base
                ok = (loc >= 0) & (loc < HALF)
                idx = jnp.where(ok, loc, HALF)
                plsc.addupdate_scatter(degv, [idx], ones_v)

            @pl.when(k + 2 < NBLKE)
            def _():
                pltpu.async_copy(dst_hbm.at[pl.ds(ebase + (k + 2) * BE, BE)],
                                 dstg[b], isem[b])

    pltpu.sync_copy(degv, parts.at[s])
    plsc.subcore_barrier()

    @pl.loop(0, SL // LANES)
    def _zs(i):
        degv[pl.ds(i * LANES, LANES)] = jnp.zeros((LANES,), jnp.float32)

    for s2 in range(NS):
        pltpu.sync_copy(parts.at[s2, pl.ds(s * SL, SL)], stg)

        @pl.loop(0, SL // LANES)
        def _acc(i):
            degv[pl.ds(i * LANES, LANES)] = (
                degv[pl.ds(i * LANES, LANES)] + stg[pl.ds(i * LANES, LANES)])

    pltpu.sync_copy(degv.at[pl.ds(0, SL)], out_hbm.at[c, pl.ds(s * SL, SL)])


def _deg_pass(dst_p):
    k = functools.partial(
        pl.kernel,
        mesh=_sc_mesh(),
        out_type=jax.ShapeDtypeStruct((NC, ACC), jnp.float32),
        scratch_types=[
            pltpu.VMEM((BE,), jnp.int32),
            pltpu.VMEM((BE,), jnp.int32),
            pltpu.VMEM((ACC,), jnp.float32),
            pltpu.VMEM((SL,), jnp.float32),
            pltpu.VMEM_SHARED((NS, ACC), jnp.float32),
            pltpu.SemaphoreType.DMA,
            pltpu.SemaphoreType.DMA,
        ],
        compiler_params=_SC_PARAMS_NL,
    )(_deg_body)
    return k(dst_p)


BE = 512
NBLKE = EPT // BE
HW = H // NC
ACC2 = 50176
RPT2 = ACC2 // NS


def _agg_body(u_hbm, src_hbm, dst_hbm, zrows_hbm, out_hbm,
              srcb0, srcb1, dstg0, dstg1, dstl0, dstl1, rows0, rows1, zv, acc,
              gsem0, gsem1, ssem0, ssem1, isem0, isem1):
    c = lax.axis_index("c")
    s = lax.axis_index("s")
    ebase = s * EPT

    srcb = (srcb0, srcb1)
    dstg = (dstg0, dstg1)
    dstl = (dstl0, dstl1)
    rows = (rows0, rows1)
    gsem = (gsem0, gsem1)
    ssem = (ssem0, ssem1)
    isem = (isem0, isem1)

    pltpu.sync_copy(zrows_hbm, zv)
    for j in range(RPT2 // ZCH):
        pltpu.sync_copy(zv, acc.at[pl.ds(s * RPT2 + j * ZCH, ZCH)])
    plsc.subcore_barrier()

    def load_block(k, kb):
        off = c * EP + ebase + k * BE
        pltpu.async_copy(src_hbm.at[pl.ds(off, BE)], srcb[kb], isem[kb])
        pltpu.async_copy(dst_hbm.at[pl.ds(ebase + k * BE, BE)], dstg[kb], isem[kb])

    def wait_block(k, kb):
        off = c * EP + ebase + k * BE
        pltpu.make_async_copy(src_hbm.at[pl.ds(off, BE)], srcb[kb], isem[kb]).wait()
        pltpu.make_async_copy(dst_hbm.at[pl.ds(ebase + k * BE, BE)], dstg[kb], isem[kb]).wait()

    def prep_chunk(jn, kbn, q):
        for t in range(CH // LANES):
            dstl[q][pl.ds(t * LANES, LANES)] = (
                dstg[kbn][pl.ds(jn * CH + t * LANES, LANES)])
        pltpu.async_copy(u_hbm.at[srcb[kbn].at[pl.ds(jn * CH, CH)]],
                         rows[q], gsem[q])

    def step(k, j, kb, skip_wait_s=False, prep_next=True, load_next2=None):
        p = j % 2
        q = 1 - p
        if not skip_wait_s:
            pltpu.make_async_copy(rows[q], acc.at[dstl[q]], ssem[q]).wait()
        if prep_next:
            if j == 3:
                wait_block(k + 1, 1 - kb)
                prep_chunk(0, 1 - kb, q)
            else:
                prep_chunk(j + 1, kb, q)
        pltpu.make_async_copy(u_hbm.at[srcb[kb].at[pl.ds(j * CH, CH)]],
                              rows[p], gsem[p]).wait()
        pltpu.async_copy(rows[p], acc.at[dstl[p]], ssem[p], add=True)
        if load_next2 is not None:
            load_next2()

    pltpu.sync_copy(src_hbm.at[pl.ds(c * EP + ebase, BE)], srcb0)
    pltpu.sync_copy(dst_hbm.at[pl.ds(ebase, BE)], dstg0)
    load_block(1, 1)
    prep_chunk(0, 0, 0)
    step(0, 0, 0, skip_wait_s=True)
    step(0, 1, 0)
    step(0, 2, 0)
    step(0, 3, 0, load_next2=lambda: load_block(2, 0))

    @pl.loop(0, (NBLKE - 2) // 2)
    def _main(m):
        for b in range(2):
            k = 1 + 2 * m + b
            kb = (1 + b) % 2
            step(k, 0, kb)
            step(k, 1, kb)
            step(k, 2, kb)

            def _ld(k=k, kb=kb):
                @pl.when(k + 2 < NBLKE)
                def _():
                    load_block(k + 2, kb)
            step(k, 3, kb, load_next2=_ld)

    kl = NBLKE - 1
    kbl = kl % 2
    step(kl, 0, kbl)
    step(kl, 1, kbl)
    step(kl, 2, kbl)
    step(kl, 3, kbl, prep_next=False)
    pltpu.make_async_copy(rows[1], acc.at[dstl[1]], ssem[1]).wait()

    plsc.subcore_barrier()
    pltpu.sync_copy(acc.at[pl.ds(s * RPT2, RPT2)],
                    out_hbm.at[c, pl.ds(s * RPT2, RPT2)])


def _agg_pass(u2, srcx, dst_p, zrows):
    k = functools.partial(
        pl.kernel,
        mesh=_sc_mesh(),
        out_type=jax.ShapeDtypeStruct((NC, ACC2, HW), jnp.float32),
        scratch_types=[
            pltpu.VMEM((BE,), jnp.int32),
            pltpu.VMEM((BE,), jnp.int32),
            pltpu.VMEM((BE,), jnp.int32),
            pltpu.VMEM((BE,), jnp.int32),
            pltpu.VMEM((CH,), jnp.int32),
            pltpu.VMEM((CH,), jnp.int32),
            pltpu.VMEM((CH, HW), jnp.float32),
            pltpu.VMEM((CH, HW), jnp.float32),
            pltpu.VMEM((ZCH, HW), jnp.float32),
            pltpu.VMEM_SHARED((ACC2, HW), jnp.float32),
            pltpu.SemaphoreType.DMA,
            pltpu.SemaphoreType.DMA,
            pltpu.SemaphoreType.DMA,
            pltpu.SemaphoreType.DMA,
            pltpu.SemaphoreType.DMA,
            pltpu.SemaphoreType.DMA,
        ],
        compiler_params=_SC_PARAMS,
    )(_agg_body)
    return k(u2, srcx, dst_p, zrows)


BLK = 2000
NBLK = N // BLK


def _mma_body(x_ref, w_ref, z_ref):
    z_ref[...] = jnp.dot(x_ref[...], w_ref[...],
                         preferred_element_type=jnp.float32)


def _mma(x, W1):
    return pl.pallas_call(
        _mma_body,
        grid=(NBLK,),
        in_specs=[
            pl.BlockSpec((BLK, D), lambda i: (i, 0)),
            pl.BlockSpec((D, H), lambda i: (0, 0)),
        ],
        out_specs=pl.BlockSpec((BLK, H), lambda i: (i, 0)),
        out_shape=jax.ShapeDtypeStruct((N, H), jnp.float32),
    )(x, W1)


def _scale_body(z_ref, deg_ref, u_ref, dinv_ref):
    deg = deg_ref[...] + 1.0
    dinv = lax.rsqrt(deg)
    u_ref[...] = z_ref[...] * dinv
    dinv_ref[...] = dinv


def _scale(z, deg):
    return pl.pallas_call(
        _scale_body,
        grid=(NBLK,),
        in_specs=[
            pl.BlockSpec((BLK, H), lambda i: (i, 0)),
            pl.BlockSpec((BLK, 1), lambda i: (i, 0)),
        ],
        out_specs=[
            pl.BlockSpec((BLK, H), lambda i: (i, 0)),
            pl.BlockSpec((BLK, 1), lambda i: (i, 0)),
        ],
        out_shape=[
            jax.ShapeDtypeStruct((N, H), jnp.float32),
            jax.ShapeDtypeStruct((N, 1), jnp.float32),
        ],
    )(z, deg)


def _layer_body(u_ref, agg_ref, dinv_ref, b_ref, w_ref, un_ref):
    dinv = dinv_ref[...]
    agg = jnp.concatenate([agg_ref[0], agg_ref[1]], axis=-1)
    h = jnp.maximum(dinv * (u_ref[...] + agg) + b_ref[...], 0.0)
    un_ref[...] = jnp.dot(h, w_ref[...], preferred_element_type=jnp.float32) * dinv


def _layer(u, agg, dinv, b, Wn):
    return pl.pallas_call(
        _layer_body,
        grid=(NBLK,),
        in_specs=[
            pl.BlockSpec((BLK, H), lambda i: (i, 0)),
            pl.BlockSpec((NC, BLK, HW), lambda i: (0, i, 0)),
            pl.BlockSpec((BLK, 1), lambda i: (i, 0)),
            pl.BlockSpec((1, H), lambda i: (0, 0)),
            pl.BlockSpec((H, H), lambda i: (0, 0)),
        ],
        out_specs=pl.BlockSpec((BLK, H), lambda i: (i, 0)),
        out_shape=jax.ShapeDtypeStruct((N, H), jnp.float32),
    )(u, agg, dinv, b, Wn)


def _head_body(u_ref, agg_ref, dinv_ref, b_ref, batch_ref, wfc_ref, bfc_ref,
               logits_ref, emb_ref, sums, cnt):
    i = pl.program_id(0)

    @pl.when(i == 0)
    def _():
        sums[...] = jnp.zeros_like(sums)
        cnt[...] = jnp.zeros_like(cnt)

    dinv = dinv_ref[...]
    agg = jnp.concatenate([agg_ref[0], agg_ref[1]], axis=-1)
    h = jnp.maximum(dinv * (u_ref[...] + agg) + b_ref[...], 0.0)
    bvec = batch_ref[...].reshape(1, BLK)
    gids = lax.broadcasted_iota(jnp.int32, (G, BLK), 0)
    mask = (gids == bvec).astype(jnp.float32)
    sums[...] += jnp.dot(mask, h, preferred_element_type=jnp.float32)
    cnt[...] += jnp.sum(mask, axis=1, keepdims=True)

    @pl.when(i == NBLK - 1)
    def _():
        emb = sums[...] / jnp.maximum(cnt[...], 1.0)
        emb_ref[...] = emb
        logits_ref[...] = (
            jnp.dot(emb, wfc_ref[...], preferred_element_type=jnp.float32)
            + bfc_ref[...])


def _head(u, agg, dinv, b, batch3, Wfc, bfc):
    return pl.pallas_call(
        _head_body,
        grid=(NBLK,),
        in_specs=[
            pl.BlockSpec((BLK, H), lambda i: (i, 0)),
            pl.BlockSpec((NC, BLK, HW), lambda i: (0, i, 0)),
            pl.BlockSpec((BLK, 1), lambda i: (i, 0)),
            pl.BlockSpec((1, H), lambda i: (0, 0)),
            pl.BlockSpec((1, 1, BLK), lambda i: (i, 0, 0)),
            pl.BlockSpec((H, 1), lambda i: (0, 0)),
            pl.BlockSpec((1, 1), lambda i: (0, 0)),
        ],
        out_specs=[
            pl.BlockSpec((G, 1), lambda i: (0, 0)),
            pl.BlockSpec((G, H), lambda i: (0, 0)),
        ],
        out_shape=[
            jax.ShapeDtypeStruct((G, 1), jnp.float32),
            jax.ShapeDtypeStruct((G, H), jnp.float32),
        ],
        scratch_shapes=[
            pltpu.VMEM((G, H), jnp.float32),
            pltpu.VMEM((G, 1), jnp.float32),
        ],
    )(u, agg, dinv, b, batch3, Wfc, bfc)


def kernel(x, edge_index, batch, W1, b1, W2, b2, W3, b3, Wfc, bfc):
    src = edge_index[0]
    dst = edge_index[1]
    pad = EP - E
    src_p = jnp.concatenate([src, jnp.zeros((pad,), jnp.int32)])
    dst_p = jnp.concatenate([dst, jnp.full((pad,), N, jnp.int32)])
    srcx = jnp.concatenate([2 * src_p, 2 * src_p + 1])

    zrows = jnp.zeros((ZCH, HW), jnp.float32)

    z1 = _mma(x, W1)
    degh = _deg_pass(dst_p)
    deg = jnp.concatenate([degh[0, :HALF], degh[1, :HALF]], axis=0)[:, None]

    u1, dinv = _scale(z1, deg)
    agg1 = _agg_pass(u1.reshape(2 * N, HW), srcx, dst_p, zrows)
    u2 = _layer(u1, agg1, dinv, b1.reshape(1, H), W2)
    agg2 = _agg_pass(u2.reshape(2 * N, HW), srcx, dst_p, zrows)
    u3 = _layer(u2, agg2, dinv, b2.reshape(1, H), W3)
    agg3 = _agg_pass(u3.reshape(2 * N, HW), srcx, dst_p, zrows)

    batch3 = batch.reshape(NBLK, 1, BLK)
    logits, emb = _head(u3, agg3, dinv, b3.reshape(1, H), batch3,
                        Wfc, bfc.reshape(1, 1))
    return (logits, emb)


# speedup vs baseline: 1.0047x; 1.0047x over previous
import functools

import jax
import jax.numpy as jnp
from jax import lax
from jax.experimental import pallas as pl
from jax.experimental.pallas import tpu as pltpu
from jax.experimental.pallas import tpu_sc as plsc

N = 50000
E = 800000
D = 128
H = 64
G = 128

NC = 2
NS = 16
LANES = 16

HALF = N // NC
ACC = 25088
ROWS_PER_TEC = ACC // NS
ZCH = 112
CH = 128
EPT = 50176
EP = EPT * NS
NCHUNK = EPT // CH

_sc_mesh = lambda: plsc.VectorSubcoreMesh(core_axis_name="c", subcore_axis_name="s")
_SC_PARAMS = pltpu.CompilerParams(use_tc_tiling_on_sc=False)
_SC_PARAMS_NL = pltpu.CompilerParams(use_tc_tiling_on_sc=False,
                                     needs_layout_passes=False)


BE = 512
NBLKE = EPT // BE
CPB = BE // CH
DW = 16


SL = ACC // NS


def _deg_body(dst_hbm, out_hbm, dstg0, dstg1, degv, stg, parts, isem0, isem1):
    c = lax.axis_index("c")
    s = lax.axis_index("s")
    base = c * HALF
    ebase = s * EPT
    dstg = (dstg0, dstg1)
    isem = (isem0, isem1)

    @pl.loop(0, ACC // LANES)
    def _z(i):
        degv[pl.ds(i * LANES, LANES)] = jnp.zeros((LANES,), jnp.float32)

    pltpu.sync_copy(dst_hbm.at[pl.ds(ebase, BE)], dstg0)
    pltpu.async_copy(dst_hbm.at[pl.ds(ebase + BE, BE)], dstg1, isem1)

    ones_v = jnp.ones((LANES,), jnp.float32)

    @pl.loop(0, NBLKE // 2)
    def _blocks(m):
        for b in range(2):
            k = 2 * m + b

            @pl.when(k > 0)
            def _():
                pltpu.make_async_copy(dst_hbm.at[pl.ds(ebase + k * BE, BE)],
                                      dstg[b], isem[b]).wait()

            for g in range(BE // LANES):
                v = dstg[b][pl.ds(g * LANES, LANES)]
                loc = v - base
                ok = (loc >= 0) & (loc < HALF)
                idx = jnp.where(ok, loc, HALF)
                plsc.addupdate_scatter(degv, [idx], ones_v)

            @pl.when(k + 2 < NBLKE)
            def _():
                pltpu.async_copy(dst_hbm.at[pl.ds(ebase + (k + 2) * BE, BE)],
                                 dstg[b], isem[b])

    pltpu.sync_copy(degv, parts.at[s])
    plsc.subcore_barrier()

    @pl.loop(0, SL // LANES)
    def _zs(i):
        degv[pl.ds(i * LANES, LANES)] = jnp.zeros((LANES,), jnp.float32)

    for s2 in range(NS):
        pltpu.sync_copy(parts.at[s2, pl.ds(s * SL, SL)], stg)

        @pl.loop(0, SL // LANES)
        def _acc(i):
            degv[pl.ds(i * LANES, LANES)] = (
                degv[pl.ds(i * LANES, LANES)] + stg[pl.ds(i * LANES, LANES)])

    pltpu.sync_copy(degv.at[pl.ds(0, SL)], out_hbm.at[c, pl.ds(s * SL, SL)])


def _deg_pass(dst_p):
    k = functools.partial(
        pl.kernel,
        mesh=_sc_mesh(),
        out_type=jax.ShapeDtypeStruct((NC, ACC), jnp.float32),
        scratch_types=[
            pltpu.VMEM((BE,), jnp.int32),
            pltpu.VMEM((BE,), jnp.int32),
            pltpu.VMEM((ACC,), jnp.float32),
            pltpu.VMEM((SL,), jnp.float32),
            pltpu.VMEM_SHARED((NS, ACC), jnp.float32),
            pltpu.SemaphoreType.DMA,
            pltpu.SemaphoreType.DMA,
        ],
        compiler_params=_SC_PARAMS_NL,
    )(_deg_body)
    return k(dst_p)


BE = 512
NBLKE = EPT // BE
HW = H // NC
ACC2 = 50176
RPT2 = ACC2 // NS


def _agg_body(u_hbm, src_hbm, dst_hbm, zrows_hbm, out_hbm,
              srcb0, srcb1, dstg0, dstg1, dstl0, dstl1, rows0, rows1, zv, acc,
              gsem0, gsem1, ssem0, ssem1, isem0, isem1):
    c = lax.axis_index("c")
    s = lax.axis_index("s")
    ebase = s * EPT

    srcb = (srcb0, srcb1)
    dstg = (dstg0, dstg1)
    dstl = (dstl0, dstl1)
    rows = (rows0, rows1)
    gsem = (gsem0, gsem1)
    ssem = (ssem0, ssem1)
    isem = (isem0, isem1)

    pltpu.sync_copy(zrows_hbm, zv)
    for j in range(RPT2 // ZCH):
        pltpu.sync_copy(zv, acc.at[pl.ds(s * RPT2 + j * ZCH, ZCH)])
    plsc.subcore_barrier()

    def load_block(k, kb):
        off = c * EP + ebase + k * BE
        pltpu.async_copy(src_hbm.at[pl.ds(off, BE)], srcb[kb], isem[kb])
        pltpu.async_copy(dst_hbm.at[pl.ds(ebase + k * BE, BE)], dstg[kb], isem[kb])

    def wait_block(k, kb):
        off = c * EP + ebase + k * BE
        pltpu.make_async_copy(src_hbm.at[pl.ds(off, BE)], srcb[kb], isem[kb]).wait()
        pltpu.make_async_copy(dst_hbm.at[pl.ds(ebase + k * BE, BE)], dstg[kb], isem[kb]).wait()

    def prep_chunk(jn, kbn, q):
        for t in range(CH // LANES):
            dstl[q][pl.ds(t * LANES, LANES)] = (
                dstg[kbn][pl.ds(jn * CH + t * LANES, LANES)])
        pltpu.async_copy(u_hbm.at[srcb[kbn].at[pl.ds(jn * CH, CH)]],
                         rows[q], gsem[q])

    def step(k, j, kb, skip_wait_s=False, prep_next=True, load_next2=None):
        p = j % 2
        q = 1 - p
        if not skip_wait_s:
            pltpu.make_async_copy(rows[q], acc.at[dstl[q]], ssem[q]).wait()
        if prep_next:
            if j == 3:
                wait_block(k + 1, 1 - kb)
                prep_chunk(0, 1 - kb, q)
            else:
                prep_chunk(j + 1, kb, q)
        pltpu.make_async_copy(u_hbm.at[srcb[kb].at[pl.ds(j * CH, CH)]],
                              rows[p], gsem[p]).wait()
        pltpu.async_copy(rows[p], acc.at[dstl[p]], ssem[p], add=True)
        if load_next2 is not None:
            load_next2()

    pltpu.sync_copy(src_hbm.at[pl.ds(c * EP + ebase, BE)], srcb0)
    pltpu.sync_copy(dst_hbm.at[pl.ds(ebase, BE)], dstg0)
    load_block(1, 1)
    prep_chunk(0, 0, 0)
    step(0, 0, 0, skip_wait_s=True)
    step(0, 1, 0)
    step(0, 2, 0)
    step(0, 3, 0, load_next2=lambda: load_block(2, 0))

    @pl.loop(0, (NBLKE - 2) // 2)
    def _main(m):
        for b in range(2):
            k = 1 + 2 * m + b
            kb = (1 + b) % 2
            step(k, 0, kb)
            step(k, 1, kb)
            step(k, 2, kb)

            def _ld(k=k, kb=kb):
                @pl.when(k + 2 < NBLKE)
                def _():
                    load_block(k + 2, kb)
            step(k, 3, kb, load_next2=_ld)

    kl = NBLKE - 1
    kbl = kl % 2
    step(kl, 0, kbl)
    step(kl, 1, kbl)
    step(kl, 2, kbl)
    step(kl, 3, kbl, prep_next=False)
    pltpu.make_async_copy(rows[1], acc.at[dstl[1]], ssem[1]).wait()

    plsc.subcore_barrier()
    pltpu.sync_copy(acc.at[pl.ds(s * RPT2, RPT2)],
                    out_hbm.at[c, pl.ds(s * RPT2, RPT2)])


def _agg_pass(u2, srcx, dst_p, zrows):
    k = functools.partial(
        pl.kernel,
        mesh=_sc_mesh(),
        out_type=jax.ShapeDtypeStruct((NC, ACC2, HW), jnp.float32),
        scratch_types=[
            pltpu.VMEM((BE,), jnp.int32),
            pltpu.VMEM((BE,), jnp.int32),
            pltpu.VMEM((BE,), jnp.int32),
            pltpu.VMEM((BE,), jnp.int32),
            pltpu.VMEM((CH,), jnp.int32),
            pltpu.VMEM((CH,), jnp.int32),
            pltpu.VMEM((CH, HW), jnp.float32),
            pltpu.VMEM((CH, HW), jnp.float32),
            pltpu.VMEM((ZCH, HW), jnp.float32),
            pltpu.VMEM_SHARED((ACC2, HW), jnp.float32),
            pltpu.SemaphoreType.DMA,
            pltpu.SemaphoreType.DMA,
            pltpu.SemaphoreType.DMA,
            pltpu.SemaphoreType.DMA,
            pltpu.SemaphoreType.DMA,
            pltpu.SemaphoreType.DMA,
        ],
        compiler_params=_SC_PARAMS,
    )(_agg_body)
    return k(u2, srcx, dst_p, zrows)


BLK = 2000
NBLK = N // BLK


def _mm1_body(x_ref, w_ref, deg_ref, u_ref, dinv_ref):
    deg = deg_ref[...] + 1.0
    dinv = lax.rsqrt(deg)
    z = jnp.dot(x_ref[...], w_ref[...], preferred_element_type=jnp.float32)
    u_ref[...] = z * dinv
    dinv_ref[...] = dinv


def _mm1(x, W1, deg):
    return pl.pallas_call(
        _mm1_body,
        grid=(NBLK,),
        in_specs=[
            pl.BlockSpec((BLK, D), lambda i: (i, 0)),
            pl.BlockSpec((D, H), lambda i: (0, 0)),
            pl.BlockSpec((BLK, 1), lambda i: (i, 0)),
        ],
        out_specs=[
            pl.BlockSpec((BLK, H), lambda i: (i, 0)),
            pl.BlockSpec((BLK, 1), lambda i: (i, 0)),
        ],
        out_shape=[
            jax.ShapeDtypeStruct((N, H), jnp.float32),
            jax.ShapeDtypeStruct((N, 1), jnp.float32),
        ],
    )(x, W1, deg)


def _layer_body(u_ref, agg_ref, dinv_ref, b_ref, w_ref, un_ref):
    dinv = dinv_ref[...]
    agg = jnp.concatenate([agg_ref[0], agg_ref[1]], axis=-1)
    h = jnp.maximum(dinv * (u_ref[...] + agg) + b_ref[...], 0.0)
    un_ref[...] = jnp.dot(h, w_ref[...], preferred_element_type=jnp.float32) * dinv


def _layer(u, agg, dinv, b, Wn):
    return pl.pallas_call(
        _layer_body,
        grid=(NBLK,),
        in_specs=[
            pl.BlockSpec((BLK, H), lambda i: (i, 0)),
            pl.BlockSpec((NC, BLK, HW), lambda i: (0, i, 0)),
            pl.BlockSpec((BLK, 1), lambda i: (i, 0)),
            pl.BlockSpec((1, H), lambda i: (0, 0)),
            pl.BlockSpec((H, H), lambda i: (0, 0)),
        ],
        out_specs=pl.BlockSpec((BLK, H), lambda i: (i, 0)),
        out_shape=jax.ShapeDtypeStruct((N, H), jnp.float32),
    )(u, agg, dinv, b, Wn)


def _head_body(u_ref, agg_ref, dinv_ref, b_ref, batch_ref, wfc_ref, bfc_ref,
               logits_ref, emb_ref, sums, cnt):
    i = pl.program_id(0)

    @pl.when(i == 0)
    def _():
        sums[...] = jnp.zeros_like(sums)
        cnt[...] = jnp.zeros_like(cnt)

    dinv = dinv_ref[...]
    agg = jnp.concatenate([agg_ref[0], agg_ref[1]], axis=-1)
    h = jnp.maximum(dinv * (u_ref[...] + agg) + b_ref[...], 0.0)
    bvec = batch_ref[...].reshape(1, BLK)
    gids = lax.broadcasted_iota(jnp.int32, (G, BLK), 0)
    mask = (gids == bvec).astype(jnp.float32)
    sums[...] += jnp.dot(mask, h, preferred_element_type=jnp.float32)
    cnt[...] += jnp.sum(mask, axis=1, keepdims=True)

    @pl.when(i == NBLK - 1)
    def _():
        emb = sums[...] / jnp.maximum(cnt[...], 1.0)
        emb_ref[...] = emb
        logits_ref[...] = (
            jnp.dot(emb, wfc_ref[...], preferred_element_type=jnp.float32)
            + bfc_ref[...])


def _head(u, agg, dinv, b, batch3, Wfc, bfc):
    return pl.pallas_call(
        _head_body,
        grid=(NBLK,),
        in_specs=[
            pl.BlockSpec((BLK, H), lambda i: (i, 0)),
            pl.BlockSpec((NC, BLK, HW), lambda i: (0, i, 0)),
            pl.BlockSpec((BLK, 1), lambda i: (i, 0)),
            pl.BlockSpec((1, H), lambda i: (0, 0)),
            pl.BlockSpec((1, 1, BLK), lambda i: (i, 0, 0)),
            pl.BlockSpec((H, 1), lambda i: (0, 0)),
            pl.BlockSpec((1, 1), lambda i: (0, 0)),
        ],
        out_specs=[
            pl.BlockSpec((G, 1), lambda i: (0, 0)),
            pl.BlockSpec((G, H), lambda i: (0, 0)),
        ],
        out_shape=[
            jax.ShapeDtypeStruct((G, 1), jnp.float32),
            jax.ShapeDtypeStruct((G, H), jnp.float32),
        ],
        scratch_shapes=[
            pltpu.VMEM((G, H), jnp.float32),
            pltpu.VMEM((G, 1), jnp.float32),
        ],
    )(u, agg, dinv, b, batch3, Wfc, bfc)


def kernel(x, edge_index, batch, W1, b1, W2, b2, W3, b3, Wfc, bfc):
    src = edge_index[0]
    dst = edge_index[1]
    pad = EP - E
    src_p = jnp.concatenate([src, jnp.zeros((pad,), jnp.int32)])
    dst_p = jnp.concatenate([dst, jnp.full((pad,), N, jnp.int32)])
    srcx = jnp.concatenate([2 * src_p, 2 * src_p + 1])

    zrows = jnp.zeros((ZCH, HW), jnp.float32)

    degh = _deg_pass(dst_p)
    deg = jnp.concatenate([degh[0, :HALF], degh[1, :HALF]], axis=0)[:, None]

    u1, dinv = _mm1(x, W1, deg)
    agg1 = _agg_pass(u1.reshape(2 * N, HW), srcx, dst_p, zrows)
    u2 = _layer(u1, agg1, dinv, b1.reshape(1, H), W2)
    agg2 = _agg_pass(u2.reshape(2 * N, HW), srcx, dst_p, zrows)
    u3 = _layer(u2, agg2, dinv, b2.reshape(1, H), W3)
    agg3 = _agg_pass(u3.reshape(2 * N, HW), srcx, dst_p, zrows)

    batch3 = batch.reshape(NBLK, 1, BLK)
    logits, emb = _head(u3, agg3, dinv, b3.reshape(1, H), batch3,
                        Wfc, bfc.reshape(1, 1))
    return (logits, emb)
